# Initial kernel scaffold; baseline (speedup 1.0000x reference)
import jax
import jax.numpy as jnp
from jax.experimental import pallas as pl


def kernel(x, edge_index, W):
    raise NotImplementedError("write your pallas kernel here")


# speedup vs baseline: 4.2184x; 4.2184x over previous
import functools

import jax
import jax.numpy as jnp
from jax import lax
from jax.experimental import pallas as pl
from jax.experimental.pallas import tpu as pltpu
from jax.experimental.pallas import tpu_sc as plsc

NC = 2
NS = 16
NW = NC * NS
K = 128
L = 16


def _sc_aggregate(x_pad, src_flat, dst_flat):
    n_pad, d = x_pad.shape
    n_chunks = src_flat.shape[0] // (NW * K)
    rows_per_tile = n_pad // NS
    mesh = plsc.VectorSubcoreMesh(
        core_axis_name="c", subcore_axis_name="s", num_cores=NC, num_subcores=NS
    )

    @functools.partial(
        pl.kernel,
        out_type=[
            jax.ShapeDtypeStruct((NC, n_pad, d), jnp.float32),
            jax.ShapeDtypeStruct((NW * n_pad,), jnp.float32),
        ],
        mesh=mesh,
        compiler_params=pltpu.CompilerParams(needs_layout_passes=False),
        scratch_types=[
            pltpu.VMEM((K,), jnp.int32),
            pltpu.VMEM((K,), jnp.int32),
            pltpu.VMEM((K, d), jnp.float32),
            pltpu.VMEM((n_pad,), jnp.float32),
            pltpu.VMEM_SHARED((n_pad, d), jnp.float32),
            pltpu.SemaphoreType.DMA,
        ],
    )
    def sc_kernel(x_hbm, src_hbm, dst_hbm, zacc_hbm,
                  out_hbm, deg_hbm,
                  src_v, dst_v, rows_v, hist_v, acc_sh, sem):
        c = lax.axis_index("c")
        s = lax.axis_index("s")
        wid = c * NS + s
        rows = pl.ds(s * rows_per_tile, rows_per_tile)
        pltpu.sync_copy(zacc_hbm.at[rows], acc_sh.at[rows])

        def zero_body(i, carry):
            hist_v[pl.ds(i * L, L)] = jnp.zeros((L,), jnp.float32)
            return carry

        lax.fori_loop(0, n_pad // L, zero_body, 0)
        plsc.subcore_barrier()

        def body(j, carry):
            off = pl.multiple_of((wid * n_chunks + j) * K, K)
            pltpu.sync_copy(src_hbm.at[pl.ds(off, K)], src_v)
            pltpu.sync_copy(dst_hbm.at[pl.ds(off, K)], dst_v)
            gather = pltpu.async_copy(x_hbm.at[src_v], rows_v, sem)
            for t in range(K // L):
                idx = src_v[pl.ds(t * L, L)]
                cnt, last = plsc.scan_count(idx)
                plsc.addupdate_scatter(
                    hist_v, [idx], cnt.astype(jnp.float32), mask=last
                )
            gather.wait()
            pltpu.sync_copy(rows_v, acc_sh.at[dst_v], add=True)
            return carry

        lax.fori_loop(0, n_chunks, body, 0)
        plsc.subcore_barrier()
        pltpu.sync_copy(acc_sh.at[rows], out_hbm.at[c, rows])
        doff = pl.multiple_of(wid * n_pad, 128)
        pltpu.sync_copy(hist_v, deg_hbm.at[pl.ds(doff, n_pad)])

    zacc = jnp.zeros((n_pad, d), jnp.float32)
    return sc_kernel(x_pad, src_flat, dst_flat, zacc)


def _tc_finish(parts, degs, W):
    _, n_pad, d = parts.shape

    def body(p_ref, dp_ref, w_ref, o_ref):
        agg = p_ref[0] + p_ref[1]
        deg = jnp.sum(dp_ref[...], axis=0)
        scale = jnp.where(deg > 0, 1.0 / deg, 0.0)
        o_ref[...] = (
            jnp.dot(agg, w_ref[...], preferred_element_type=jnp.float32)
            * scale[:, None]
        )

    return pl.pallas_call(
        body,
        out_shape=jax.ShapeDtypeStruct((n_pad, d), jnp.float32),
    )(parts, degs, W)


def kernel(x, edge_index, W):
    n, d = x.shape
    src = edge_index[0].astype(jnp.int32)
    dst = edge_index[1].astype(jnp.int32)
    e = src.shape[0]

    n_pad = -(-(n + 1) // (NS * 8)) * (NS * 8)
    e_per_w = -(-e // (NW * K)) * K
    pad = NW * e_per_w - e
    src_flat = jnp.concatenate([src, jnp.full((pad,), n, jnp.int32)])
    dst_flat = jnp.concatenate([dst, jnp.full((pad,), n, jnp.int32)])
    x_pad = jnp.zeros((n_pad, d), jnp.float32).at[:n].set(x)

    parts, deg_flat = _sc_aggregate(x_pad, src_flat, dst_flat)
    degs = deg_flat.reshape(NW, n_pad)
    h = _tc_finish(parts, degs, W)
    return h[:n]
